# baseline (speedup 1.0000x reference)
import functools

import jax
import jax.numpy as jnp
from jax import lax
from jax.experimental import pallas as pl
from jax.experimental.pallas import tpu as pltpu
from jax.experimental.pallas import tpu_sc as plsc

B = 4
S = 8192
D = 1024
NC = 2
NS = 16
NW = NC * NS
S_W = S // NW
CS = 16
NCH = S_W // CS
ROWS = B * CS


def _pos_table():
    pos = jnp.arange(S, dtype=jnp.float32)[:, None]
    i = jnp.arange(0, D, 2, dtype=jnp.float32)[None, :]
    angle = pos / jnp.power(10000.0, i / float(D))
    pe = jnp.zeros((S, D), dtype=jnp.float32)
    pe = pe.at[:, 0::2].set(jnp.sin(angle))
    pe = pe.at[:, 1::2].set(jnp.cos(angle))
    return pe


@functools.partial(
    pl.kernel,
    out_type=jax.ShapeDtypeStruct((B, S, D), jnp.float32),
    mesh=plsc.VectorSubcoreMesh(core_axis_name="c", subcore_axis_name="s"),
    scratch_types=[
        pltpu.VMEM((NCH * ROWS,), jnp.int32),
        pltpu.VMEM((ROWS, D), jnp.float32),
        pltpu.VMEM((CS, D), jnp.float32),
    ],
)
def _sc_embed(xr_hbm, tok_hbm, pos_hbm, out_hbm, idx_v, gath_v, pos_v):
    wid = lax.axis_index("s") * NC + lax.axis_index("c")
    s0 = wid * S_W
    pltpu.sync_copy(xr_hbm.at[wid], idx_v)

    @pl.loop(0, NCH)
    def _chunk(c):
        sb = s0 + c * CS
        pltpu.sync_copy(pos_hbm.at[pl.ds(sb, CS)], pos_v)
        pltpu.sync_copy(tok_hbm.at[idx_v.at[pl.ds(c * ROWS, ROWS)]], gath_v)

        @pl.loop(0, CS)
        def _row(k):
            @pl.loop(0, D, step=16)
            def _col(d):
                t = pos_v[k, pl.ds(d, 16)]
                for b in range(B):
                    plsc.addupdate(gath_v.at[b * CS + k, pl.ds(d, 16)], t)

        for b in range(B):
            pltpu.sync_copy(gath_v.at[pl.ds(b * CS, CS)],
                            out_hbm.at[b].at[pl.ds(sb, CS)])


def kernel(x, tok_table):
    pos = _pos_table()
    xr = (x.reshape(B, NW, NCH, CS)
          .transpose(1, 2, 0, 3)
          .reshape(NW, NCH * B * CS))
    return _sc_embed(xr, tok_table, pos)
